# baseline (speedup 1.0000x reference)
import jax
import jax.numpy as jnp
from jax import lax
from jax.experimental import pallas as pl
from jax.experimental.pallas import tpu as pltpu

_NN = 29
_NP = 32
_ND = 3
_INF = 6
_CTX = 2
_H = 64
_L = 4
_NORM = 100.0
_B = 4

_INTERPRET = False


def _silu(u):
    return u * jnp.tanh(u) + u


def _body(*refs):
    x0_ref, hc_ref, nm_ref, s_ref = refs[:4]
    out_ref = refs[-1]
    prefs = refs[4:-1]

    B, NP, H = _B, _NP, _H
    Bn = B * NP
    E = B * NP * NP
    f32 = jnp.float32
    npad = float(_NP - _NN)

    x0 = x0_ref[...].reshape(Bn, _ND)
    nmf = nm_ref[...].reshape(Bn, 1)
    sgrp = s_ref[...]
    ones31 = jnp.ones((_ND, 1), f32)

    hc = hc_ref[...].reshape(Bn, _INF + 1 + _CTX)
    ew = prefs[0][...]
    eb = prefs[1][...]
    h = (jnp.dot(hc, ew, preferred_element_type=f32) + eb) * nmf
    x = x0

    idx = 2
    for _l in range(_L):
        (e1a, e1b, e1d, e1bias, e2w, e2b, c1w, c1b, c2w, c2b,
         n1a, n1b, n1bias, n2w, n2b) = (p[...] for p in prefs[idx:idx + 15])
        idx += 15

        ai2 = jnp.dot(h, e1a, preferred_element_type=f32) + e1bias
        ai = ai2.reshape(B, NP, H)
        bj = jnp.dot(h, e1b, preferred_element_type=f32).reshape(B, NP, H)
        pre = (lax.broadcast_in_dim(ai, (B, NP, NP, H), (0, 1, 3)) +
               lax.broadcast_in_dim(bj, (B, NP, NP, H), (0, 2, 3))).reshape(E, H)

        x3 = x.reshape(B, NP, _ND)
        dif = (lax.broadcast_in_dim(x3, (B, NP, NP, _ND), (0, 1, 3)) -
               lax.broadcast_in_dim(x3, (B, NP, NP, _ND), (0, 2, 3))).reshape(E, _ND)
        d2 = jnp.dot(dif * dif, ones31, preferred_element_type=f32)
        pre = pre + d2 * e1d

        m = _silu(jnp.dot(_silu(pre), e2w, preferred_element_type=f32) + e2b)
        cc = _silu(jnp.dot(m, c1w, preferred_element_type=f32) + c1b)
        c = jnp.dot(cc, c2w, preferred_element_type=f32) + c2b
        w = c * lax.rsqrt(d2 + 1e-8)

        r2 = jnp.sum(x * x, axis=1, keepdims=True)
        ppre = ai2 + r2 * e1d
        mpad = _silu(jnp.dot(_silu(ppre), e2w, preferred_element_type=f32) + e2b)
        cpad = (jnp.dot(_silu(jnp.dot(mpad, c1w, preferred_element_type=f32) + c1b),
                        c2w, preferred_element_type=f32) + c2b)
        wpad = cpad * lax.rsqrt(r2 + 1e-8)

        wl = w.reshape(Bn, NP)
        w1s = jnp.sum(wl, axis=1, keepdims=True)
        wl3 = wl.reshape(B, NP, NP)
        xw = jnp.concatenate(
            [jnp.dot(wl3[b], x3[b], preferred_element_type=f32)
             for b in range(B)], axis=0)
        aggx = (x * (w1s - npad * wpad) - xw) * (1.0 / _NORM)
        x = (x + aggx) * nmf
        summ = jnp.dot(sgrp, m, preferred_element_type=f32)
        aggm = (summ - npad * mpad) * (1.0 / _NORM)

        npre = (jnp.dot(h, n1a, preferred_element_type=f32) +
                jnp.dot(aggm, n1b, preferred_element_type=f32) + n1bias)
        h = (h + jnp.dot(_silu(npre), n2w, preferred_element_type=f32) + n2b) * nmf

    ow = prefs[idx][...]
    ob = prefs[idx + 1][...]
    hout = (jnp.dot(h, ow, preferred_element_type=f32) + ob) * nmf
    hf = hout[:, :_INF]

    vel = (x - x0) * nmf
    v3 = vel.reshape(B, NP, _ND)
    nm3 = nmf.reshape(B, NP, 1)
    cnt = jnp.sum(nm3, axis=1, keepdims=True)
    mean = jnp.sum(v3 * nm3, axis=1, keepdims=True) / cnt
    v3 = (v3 - mean) * nm3
    out_ref[...] = jnp.concatenate([v3, hf.reshape(B, NP, _INF)], axis=2)


def kernel(t, xh, node_mask, edge_mask, context, params):
    bs, n, dims = xh.shape
    f32 = jnp.float32
    nm = node_mask.astype(f32)
    xm = xh * nm
    x0 = xm[:, :, :_ND]
    tcol = jnp.full((bs, n, 1), t[0], f32)
    hcat = jnp.concatenate([xm[:, :, _ND:], tcol, context], axis=2)

    pad = _NP - n
    x0p = jnp.pad(x0, ((0, 0), (0, pad), (0, 0)))
    hcp = jnp.pad(hcat, ((0, 0), (0, pad), (0, 0)))
    nmp = jnp.pad(nm, ((0, 0), (0, pad), (0, 0)))
    sgrp = jnp.kron(jnp.eye(_B * _NP, dtype=f32), jnp.ones((1, _NP), f32))

    H = _H
    plist = [params['emb'][0], params['emb'][1].reshape(1, H)]
    for l in range(_L):
        w1, b1 = params['e1_%d' % l]
        w2, b2 = params['e2_%d' % l]
        cw1, cb1 = params['c1_%d' % l]
        cw2, cb2 = params['c2_%d' % l]
        nw1, nb1 = params['n1_%d' % l]
        nw2, nb2 = params['n2_%d' % l]
        plist += [0.5 * w1[:H], 0.5 * w1[H:2 * H], 0.5 * w1[2 * H:],
                  0.5 * b1.reshape(1, H),
                  0.5 * w2, 0.5 * b2.reshape(1, H),
                  0.5 * cw1, 0.5 * cb1.reshape(1, H),
                  cw2, cb2.reshape(1, 1),
                  0.5 * nw1[:H], 0.5 * nw1[H:], 0.5 * nb1.reshape(1, H),
                  nw2, nb2.reshape(1, H)]
    plist += [params['out'][0], params['out'][1].reshape(1, dims)]

    data_specs = [
        pl.BlockSpec((_B, _NP, _ND), lambda i: (i, 0, 0)),
        pl.BlockSpec((_B, _NP, _INF + 1 + _CTX), lambda i: (i, 0, 0)),
        pl.BlockSpec((_B, _NP, 1), lambda i: (i, 0, 0)),
        pl.BlockSpec((_B * _NP, _B * _NP * _NP), lambda i: (0, 0)),
    ]
    param_specs = [
        pl.BlockSpec(p.shape, (lambda nd: lambda i: (0,) * nd)(p.ndim))
        for p in plist
    ]

    out = pl.pallas_call(
        _body,
        grid=(bs // _B,),
        in_specs=data_specs + param_specs,
        out_specs=pl.BlockSpec((_B, _NP, dims), lambda i: (i, 0, 0)),
        out_shape=jax.ShapeDtypeStruct((bs, _NP, dims), f32),
        compiler_params=pltpu.CompilerParams(
            dimension_semantics=("parallel",),
            vmem_limit_bytes=100 * 1024 * 1024,
        ),
        interpret=_INTERPRET,
    )(x0p, hcp, nmp, sgrp, *plist)
    return out[:, :n, :]


# speedup vs baseline: 1.0888x; 1.0888x over previous
import jax
import jax.numpy as jnp
from jax import lax
from jax.experimental import pallas as pl
from jax.experimental.pallas import tpu as pltpu

_NN = 29
_NP = 32
_ND = 3
_INF = 6
_CTX = 2
_H = 64
_L = 4
_NORM = 100.0
_B = 4

_INTERPRET = False


def _silu(u):
    return u * jnp.tanh(u) + u


def _body(*refs):
    x0_ref, hc_ref, nm_ref = refs[:3]
    out_ref = refs[-1]
    prefs = refs[3:-1]

    B, NP, H = _B, _NP, _H
    Bn = B * NP
    E = B * NP * NP
    f32 = jnp.float32
    npad = float(_NP - _NN)

    x0 = x0_ref[...].reshape(Bn, _ND)
    nmf = nm_ref[...].reshape(Bn, 1)
    ones31 = jnp.ones((_ND, 1), f32)

    hc = hc_ref[...].reshape(Bn, _INF + 1 + _CTX)
    ew = prefs[0][...]
    eb = prefs[1][...]
    h = (jnp.dot(hc, ew, preferred_element_type=f32) + eb) * nmf
    x = x0

    idx = 2
    for _l in range(_L):
        (e1a, e1b, e1d, e1bias, e2w, e2b, c1w, c1b, c2w, c2b,
         n1a, n1b, n1bias, n2w, n2b) = (p[...] for p in prefs[idx:idx + 15])
        idx += 15

        ai2 = jnp.dot(h, e1a, preferred_element_type=f32) + e1bias
        ai = ai2.reshape(B, NP, H)
        bj = jnp.dot(h, e1b, preferred_element_type=f32).reshape(B, NP, H)
        pre = (lax.broadcast_in_dim(ai, (B, NP, NP, H), (0, 1, 3)) +
               lax.broadcast_in_dim(bj, (B, NP, NP, H), (0, 2, 3))).reshape(E, H)

        x3 = x.reshape(B, NP, _ND)
        dif = (lax.broadcast_in_dim(x3, (B, NP, NP, _ND), (0, 1, 3)) -
               lax.broadcast_in_dim(x3, (B, NP, NP, _ND), (0, 2, 3))).reshape(E, _ND)
        d2 = jnp.dot(dif * dif, ones31, preferred_element_type=f32)
        pre = pre + d2 * e1d

        m = _silu(jnp.dot(_silu(pre), e2w, preferred_element_type=f32) + e2b)
        cc = _silu(jnp.dot(m, c1w, preferred_element_type=f32) + c1b)
        c = jnp.dot(cc, c2w, preferred_element_type=f32) + c2b
        w = c * lax.rsqrt(d2 + 1e-8)

        r2 = jnp.sum(x * x, axis=1, keepdims=True)
        ppre = ai2 + r2 * e1d
        mpad = _silu(jnp.dot(_silu(ppre), e2w, preferred_element_type=f32) + e2b)
        cpad = (jnp.dot(_silu(jnp.dot(mpad, c1w, preferred_element_type=f32) + c1b),
                        c2w, preferred_element_type=f32) + c2b)
        wpad = cpad * lax.rsqrt(r2 + 1e-8)

        wl = w.reshape(Bn, NP)
        w1s = jnp.sum(wl, axis=1, keepdims=True)
        wl3 = wl.reshape(B, NP, NP)
        xw = jnp.concatenate(
            [jnp.dot(wl3[b], x3[b], preferred_element_type=f32)
             for b in range(B)], axis=0)
        aggx = (x * (w1s - npad * wpad) - xw) * (1.0 / _NORM)
        x = (x + aggx) * nmf
        summ = jnp.sum(m.reshape(Bn, NP, H), axis=1)
        aggm = (summ - npad * mpad) * (1.0 / _NORM)

        npre = (jnp.dot(h, n1a, preferred_element_type=f32) +
                jnp.dot(aggm, n1b, preferred_element_type=f32) + n1bias)
        h = (h + jnp.dot(_silu(npre), n2w, preferred_element_type=f32) + n2b) * nmf

    ow = prefs[idx][...]
    ob = prefs[idx + 1][...]
    hout = (jnp.dot(h, ow, preferred_element_type=f32) + ob) * nmf
    hf = hout[:, :_INF]

    vel = (x - x0) * nmf
    v3 = vel.reshape(B, NP, _ND)
    nm3 = nmf.reshape(B, NP, 1)
    cnt = jnp.sum(nm3, axis=1, keepdims=True)
    mean = jnp.sum(v3 * nm3, axis=1, keepdims=True) / cnt
    v3 = (v3 - mean) * nm3
    out_ref[...] = jnp.concatenate([v3, hf.reshape(B, NP, _INF)], axis=2)


def kernel(t, xh, node_mask, edge_mask, context, params):
    bs, n, dims = xh.shape
    f32 = jnp.float32
    nm = node_mask.astype(f32)
    xm = xh * nm
    x0 = xm[:, :, :_ND]
    tcol = jnp.full((bs, n, 1), t[0], f32)
    hcat = jnp.concatenate([xm[:, :, _ND:], tcol, context], axis=2)

    pad = _NP - n
    x0p = jnp.pad(x0, ((0, 0), (0, pad), (0, 0)))
    hcp = jnp.pad(hcat, ((0, 0), (0, pad), (0, 0)))
    nmp = jnp.pad(nm, ((0, 0), (0, pad), (0, 0)))

    H = _H
    plist = [params['emb'][0], params['emb'][1].reshape(1, H)]
    for l in range(_L):
        w1, b1 = params['e1_%d' % l]
        w2, b2 = params['e2_%d' % l]
        cw1, cb1 = params['c1_%d' % l]
        cw2, cb2 = params['c2_%d' % l]
        nw1, nb1 = params['n1_%d' % l]
        nw2, nb2 = params['n2_%d' % l]
        plist += [0.5 * w1[:H], 0.5 * w1[H:2 * H], 0.5 * w1[2 * H:],
                  0.5 * b1.reshape(1, H),
                  0.5 * w2, 0.5 * b2.reshape(1, H),
                  0.5 * cw1, 0.5 * cb1.reshape(1, H),
                  cw2, cb2.reshape(1, 1),
                  0.5 * nw1[:H], 0.5 * nw1[H:], 0.5 * nb1.reshape(1, H),
                  nw2, nb2.reshape(1, H)]
    plist += [params['out'][0], params['out'][1].reshape(1, dims)]

    data_specs = [
        pl.BlockSpec((_B, _NP, _ND), lambda i: (i, 0, 0)),
        pl.BlockSpec((_B, _NP, _INF + 1 + _CTX), lambda i: (i, 0, 0)),
        pl.BlockSpec((_B, _NP, 1), lambda i: (i, 0, 0)),
    ]
    param_specs = [
        pl.BlockSpec(p.shape, (lambda nd: lambda i: (0,) * nd)(p.ndim))
        for p in plist
    ]

    out = pl.pallas_call(
        _body,
        grid=(bs // _B,),
        in_specs=data_specs + param_specs,
        out_specs=pl.BlockSpec((_B, _NP, dims), lambda i: (i, 0, 0)),
        out_shape=jax.ShapeDtypeStruct((bs, _NP, dims), f32),
        compiler_params=pltpu.CompilerParams(
            dimension_semantics=("parallel",),
            vmem_limit_bytes=100 * 1024 * 1024,
        ),
        interpret=_INTERPRET,
    )(x0p, hcp, nmp, *plist)
    return out[:, :n, :]


# speedup vs baseline: 1.1741x; 1.0783x over previous
import jax
import jax.numpy as jnp
from jax import lax
from jax.experimental import pallas as pl
from jax.experimental.pallas import tpu as pltpu

_NN = 29
_NP = 32
_ND = 3
_INF = 6
_CTX = 2
_H = 64
_L = 4
_NORM = 100.0
_B = 8

_INTERPRET = False


def _silu(u):
    return u * jnp.tanh(u) + u


def _body(*refs):
    x0_ref, hc_ref, nm_ref = refs[:3]
    out_ref = refs[-1]
    prefs = refs[3:-1]

    B, NP, H = _B, _NP, _H
    Bn = B * NP
    E = B * NP * NP
    f32 = jnp.float32
    npad = float(_NP - _NN)

    x0 = x0_ref[...].reshape(Bn, _ND)
    nmf = nm_ref[...].reshape(Bn, 1)
    ones31 = jnp.ones((_ND, 1), f32)

    hc = hc_ref[...].reshape(Bn, _INF + 1 + _CTX)
    ew = prefs[0][...]
    eb = prefs[1][...]
    h = (jnp.dot(hc, ew, preferred_element_type=f32) + eb) * nmf
    x = x0

    idx = 2
    for _l in range(_L):
        (e1a, e1b, e1d, e1bias, e2w, e2b, c1w, c1b, c2w, c2b,
         n1a, n1b, n1bias, n2w, n2b) = (p[...] for p in prefs[idx:idx + 15])
        idx += 15

        ai2 = jnp.dot(h, e1a, preferred_element_type=f32) + e1bias
        ai = ai2.reshape(B, NP, H)
        bj = jnp.dot(h, e1b, preferred_element_type=f32).reshape(B, NP, H)
        pre = (lax.broadcast_in_dim(ai, (B, NP, NP, H), (0, 1, 3)) +
               lax.broadcast_in_dim(bj, (B, NP, NP, H), (0, 2, 3))).reshape(E, H)

        x3 = x.reshape(B, NP, _ND)
        dif = (lax.broadcast_in_dim(x3, (B, NP, NP, _ND), (0, 1, 3)) -
               lax.broadcast_in_dim(x3, (B, NP, NP, _ND), (0, 2, 3))).reshape(E, _ND)
        d2 = jnp.dot(dif * dif, ones31, preferred_element_type=f32)
        pre = pre + d2 * e1d

        m = _silu(jnp.dot(_silu(pre), e2w, preferred_element_type=f32) + e2b)
        cc = _silu(jnp.dot(m, c1w, preferred_element_type=f32) + c1b)
        c = jnp.dot(cc, c2w, preferred_element_type=f32) + c2b
        w = c * lax.rsqrt(d2 + 1e-8)

        r2 = jnp.sum(x * x, axis=1, keepdims=True)
        ppre = ai2 + r2 * e1d
        mpad = _silu(jnp.dot(_silu(ppre), e2w, preferred_element_type=f32) + e2b)
        cpad = (jnp.dot(_silu(jnp.dot(mpad, c1w, preferred_element_type=f32) + c1b),
                        c2w, preferred_element_type=f32) + c2b)
        wpad = cpad * lax.rsqrt(r2 + 1e-8)

        wl = w.reshape(Bn, NP)
        w1s = jnp.sum(wl, axis=1, keepdims=True)
        wl3 = wl.reshape(B, NP, NP)
        xw = jnp.concatenate(
            [jnp.dot(wl3[b], x3[b], preferred_element_type=f32)
             for b in range(B)], axis=0)
        aggx = (x * (w1s - npad * wpad) - xw) * (1.0 / _NORM)
        x = (x + aggx) * nmf
        summ = jnp.sum(m.reshape(Bn, NP, H), axis=1)
        aggm = (summ - npad * mpad) * (1.0 / _NORM)

        npre = (jnp.dot(h, n1a, preferred_element_type=f32) +
                jnp.dot(aggm, n1b, preferred_element_type=f32) + n1bias)
        h = (h + jnp.dot(_silu(npre), n2w, preferred_element_type=f32) + n2b) * nmf

    ow = prefs[idx][...]
    ob = prefs[idx + 1][...]
    hout = (jnp.dot(h, ow, preferred_element_type=f32) + ob) * nmf
    hf = hout[:, :_INF]

    vel = (x - x0) * nmf
    v3 = vel.reshape(B, NP, _ND)
    nm3 = nmf.reshape(B, NP, 1)
    cnt = jnp.sum(nm3, axis=1, keepdims=True)
    mean = jnp.sum(v3 * nm3, axis=1, keepdims=True) / cnt
    v3 = (v3 - mean) * nm3
    out_ref[...] = jnp.concatenate([v3, hf.reshape(B, NP, _INF)], axis=2)


def kernel(t, xh, node_mask, edge_mask, context, params):
    bs, n, dims = xh.shape
    f32 = jnp.float32
    nm = node_mask.astype(f32)
    xm = xh * nm
    x0 = xm[:, :, :_ND]
    tcol = jnp.full((bs, n, 1), t[0], f32)
    hcat = jnp.concatenate([xm[:, :, _ND:], tcol, context], axis=2)

    pad = _NP - n
    x0p = jnp.pad(x0, ((0, 0), (0, pad), (0, 0)))
    hcp = jnp.pad(hcat, ((0, 0), (0, pad), (0, 0)))
    nmp = jnp.pad(nm, ((0, 0), (0, pad), (0, 0)))

    H = _H
    plist = [params['emb'][0], params['emb'][1].reshape(1, H)]
    for l in range(_L):
        w1, b1 = params['e1_%d' % l]
        w2, b2 = params['e2_%d' % l]
        cw1, cb1 = params['c1_%d' % l]
        cw2, cb2 = params['c2_%d' % l]
        nw1, nb1 = params['n1_%d' % l]
        nw2, nb2 = params['n2_%d' % l]
        plist += [0.5 * w1[:H], 0.5 * w1[H:2 * H], 0.5 * w1[2 * H:],
                  0.5 * b1.reshape(1, H),
                  0.5 * w2, 0.5 * b2.reshape(1, H),
                  0.5 * cw1, 0.5 * cb1.reshape(1, H),
                  cw2, cb2.reshape(1, 1),
                  0.5 * nw1[:H], 0.5 * nw1[H:], 0.5 * nb1.reshape(1, H),
                  nw2, nb2.reshape(1, H)]
    plist += [params['out'][0], params['out'][1].reshape(1, dims)]

    data_specs = [
        pl.BlockSpec((_B, _NP, _ND), lambda i: (i, 0, 0)),
        pl.BlockSpec((_B, _NP, _INF + 1 + _CTX), lambda i: (i, 0, 0)),
        pl.BlockSpec((_B, _NP, 1), lambda i: (i, 0, 0)),
    ]
    param_specs = [
        pl.BlockSpec(p.shape, (lambda nd: lambda i: (0,) * nd)(p.ndim))
        for p in plist
    ]

    out = pl.pallas_call(
        _body,
        grid=(bs // _B,),
        in_specs=data_specs + param_specs,
        out_specs=pl.BlockSpec((_B, _NP, dims), lambda i: (i, 0, 0)),
        out_shape=jax.ShapeDtypeStruct((bs, _NP, dims), f32),
        compiler_params=pltpu.CompilerParams(
            dimension_semantics=("parallel",),
            vmem_limit_bytes=100 * 1024 * 1024,
        ),
        interpret=_INTERPRET,
    )(x0p, hcp, nmp, *plist)
    return out[:, :n, :]


# speedup vs baseline: 1.1950x; 1.0178x over previous
import jax
import jax.numpy as jnp
from jax import lax
from jax.experimental import pallas as pl
from jax.experimental.pallas import tpu as pltpu

_NN = 29
_NP = 32
_ND = 3
_INF = 6
_CTX = 2
_H = 64
_L = 4
_NORM = 100.0
_B = 16

_INTERPRET = False


def _silu(u):
    return u * jnp.tanh(u) + u


def _body(*refs):
    x0_ref, hc_ref, nm_ref = refs[:3]
    out_ref = refs[-1]
    prefs = refs[3:-1]

    B, NP, H = _B, _NP, _H
    Bn = B * NP
    E = B * NP * NP
    f32 = jnp.float32
    npad = float(_NP - _NN)

    x0 = x0_ref[...].reshape(Bn, _ND)
    nmf = nm_ref[...].reshape(Bn, 1)
    ones31 = jnp.ones((_ND, 1), f32)

    hc = hc_ref[...].reshape(Bn, _INF + 1 + _CTX)
    ew = prefs[0][...]
    eb = prefs[1][...]
    h = (jnp.dot(hc, ew, preferred_element_type=f32) + eb) * nmf
    x = x0

    idx = 2
    for _l in range(_L):
        (e1a, e1b, e1d, e1bias, e2w, e2b, c1w, c1b, c2w, c2b,
         n1a, n1b, n1bias, n2w, n2b) = (p[...] for p in prefs[idx:idx + 15])
        idx += 15

        ai2 = jnp.dot(h, e1a, preferred_element_type=f32) + e1bias
        ai = ai2.reshape(B, NP, H)
        bj = jnp.dot(h, e1b, preferred_element_type=f32).reshape(B, NP, H)
        pre = (lax.broadcast_in_dim(ai, (B, NP, NP, H), (0, 1, 3)) +
               lax.broadcast_in_dim(bj, (B, NP, NP, H), (0, 2, 3))).reshape(E, H)

        x3 = x.reshape(B, NP, _ND)
        dif = (lax.broadcast_in_dim(x3, (B, NP, NP, _ND), (0, 1, 3)) -
               lax.broadcast_in_dim(x3, (B, NP, NP, _ND), (0, 2, 3))).reshape(E, _ND)
        d2 = jnp.dot(dif * dif, ones31, preferred_element_type=f32)
        pre = pre + d2 * e1d

        m = _silu(jnp.dot(_silu(pre), e2w, preferred_element_type=f32) + e2b)
        cc = _silu(jnp.dot(m, c1w, preferred_element_type=f32) + c1b)
        c = jnp.dot(cc, c2w, preferred_element_type=f32) + c2b
        w = c * lax.rsqrt(d2 + 1e-8)

        r2 = jnp.sum(x * x, axis=1, keepdims=True)
        ppre = ai2 + r2 * e1d
        mpad = _silu(jnp.dot(_silu(ppre), e2w, preferred_element_type=f32) + e2b)
        cpad = (jnp.dot(_silu(jnp.dot(mpad, c1w, preferred_element_type=f32) + c1b),
                        c2w, preferred_element_type=f32) + c2b)
        wpad = cpad * lax.rsqrt(r2 + 1e-8)

        wl = w.reshape(Bn, NP)
        w1s = jnp.sum(wl, axis=1, keepdims=True)
        wl3 = wl.reshape(B, NP, NP)
        xw = jnp.concatenate(
            [jnp.dot(wl3[b], x3[b], preferred_element_type=f32)
             for b in range(B)], axis=0)
        aggx = (x * (w1s - npad * wpad) - xw) * (1.0 / _NORM)
        x = (x + aggx) * nmf
        summ = jnp.sum(m.reshape(Bn, NP, H), axis=1)
        aggm = (summ - npad * mpad) * (1.0 / _NORM)

        npre = (jnp.dot(h, n1a, preferred_element_type=f32) +
                jnp.dot(aggm, n1b, preferred_element_type=f32) + n1bias)
        h = (h + jnp.dot(_silu(npre), n2w, preferred_element_type=f32) + n2b) * nmf

    ow = prefs[idx][...]
    ob = prefs[idx + 1][...]
    hout = (jnp.dot(h, ow, preferred_element_type=f32) + ob) * nmf
    hf = hout[:, :_INF]

    vel = (x - x0) * nmf
    v3 = vel.reshape(B, NP, _ND)
    nm3 = nmf.reshape(B, NP, 1)
    cnt = jnp.sum(nm3, axis=1, keepdims=True)
    mean = jnp.sum(v3 * nm3, axis=1, keepdims=True) / cnt
    v3 = (v3 - mean) * nm3
    out_ref[...] = jnp.concatenate([v3, hf.reshape(B, NP, _INF)], axis=2)


def kernel(t, xh, node_mask, edge_mask, context, params):
    bs, n, dims = xh.shape
    f32 = jnp.float32
    nm = node_mask.astype(f32)
    xm = xh * nm
    x0 = xm[:, :, :_ND]
    tcol = jnp.full((bs, n, 1), t[0], f32)
    hcat = jnp.concatenate([xm[:, :, _ND:], tcol, context], axis=2)

    pad = _NP - n
    x0p = jnp.pad(x0, ((0, 0), (0, pad), (0, 0)))
    hcp = jnp.pad(hcat, ((0, 0), (0, pad), (0, 0)))
    nmp = jnp.pad(nm, ((0, 0), (0, pad), (0, 0)))

    H = _H
    plist = [params['emb'][0], params['emb'][1].reshape(1, H)]
    for l in range(_L):
        w1, b1 = params['e1_%d' % l]
        w2, b2 = params['e2_%d' % l]
        cw1, cb1 = params['c1_%d' % l]
        cw2, cb2 = params['c2_%d' % l]
        nw1, nb1 = params['n1_%d' % l]
        nw2, nb2 = params['n2_%d' % l]
        plist += [0.5 * w1[:H], 0.5 * w1[H:2 * H], 0.5 * w1[2 * H:],
                  0.5 * b1.reshape(1, H),
                  0.5 * w2, 0.5 * b2.reshape(1, H),
                  0.5 * cw1, 0.5 * cb1.reshape(1, H),
                  cw2, cb2.reshape(1, 1),
                  0.5 * nw1[:H], 0.5 * nw1[H:], 0.5 * nb1.reshape(1, H),
                  nw2, nb2.reshape(1, H)]
    plist += [params['out'][0], params['out'][1].reshape(1, dims)]

    data_specs = [
        pl.BlockSpec((_B, _NP, _ND), lambda i: (i, 0, 0)),
        pl.BlockSpec((_B, _NP, _INF + 1 + _CTX), lambda i: (i, 0, 0)),
        pl.BlockSpec((_B, _NP, 1), lambda i: (i, 0, 0)),
    ]
    param_specs = [
        pl.BlockSpec(p.shape, (lambda nd: lambda i: (0,) * nd)(p.ndim))
        for p in plist
    ]

    out = pl.pallas_call(
        _body,
        grid=(bs // _B,),
        in_specs=data_specs + param_specs,
        out_specs=pl.BlockSpec((_B, _NP, dims), lambda i: (i, 0, 0)),
        out_shape=jax.ShapeDtypeStruct((bs, _NP, dims), f32),
        compiler_params=pltpu.CompilerParams(
            dimension_semantics=("parallel",),
            vmem_limit_bytes=100 * 1024 * 1024,
        ),
        interpret=_INTERPRET,
    )(x0p, hcp, nmp, *plist)
    return out[:, :n, :]
